# baseline (speedup 1.0000x reference)
import jax
import jax.numpy as jnp
from jax import lax
from jax.experimental import pallas as pl
from jax.experimental.pallas import tpu as pltpu
from jax.experimental.pallas import tpu_sc as plsc

_N_NODES = 100000
_N_EDGES = 6400000
_NUM_ET = 38
_HID = 128
_TBL = 640
_LANES = 128
_ROWS = _N_EDGES // _LANES
_CHUNK = 8
_NCHUNK = _ROWS // _CHUNK
_NW = 32
_ZC = 800
_NZC = _N_NODES // _ZC


def _table_body(w1_ref, b1_ref, w2_ref, b2_ref, o_ref):
    row = lax.broadcasted_iota(jnp.int32, (_TBL, _HID), 0)
    col = lax.broadcasted_iota(jnp.int32, (_TBL, _HID), 1)
    et = row // 16
    ht = (row // 4) % 4
    tt = row % 4
    feat = ((col == et)
            | ((col >= _NUM_ET) & (col < _NUM_ET + 4) & (col - _NUM_ET == ht))
            | ((col >= _NUM_ET + 4) & (col < _NUM_ET + 8)
               & (col - (_NUM_ET + 4) == tt))).astype(jnp.float32)
    h = jnp.dot(feat, w1_ref[...], preferred_element_type=jnp.float32,
                precision="highest") + b1_ref[...]
    g = jax.nn.gelu(h)
    o = jnp.dot(g, w2_ref[...], preferred_element_type=jnp.float32,
                precision="highest") + b2_ref[...]
    o_ref[...] = jax.nn.sigmoid(o)


def _make_table(W1, b1, W2, b2):
    w1p = jnp.zeros((_HID, _HID), jnp.float32).at[: W1.shape[0]].set(W1)
    w2p = jnp.zeros((_HID, _HID), jnp.float32).at[:, :1].set(W2)
    b1r = b1.reshape(1, _HID)
    b2r = jnp.broadcast_to(b2.reshape(1, 1), (1, _HID))
    out = pl.pallas_call(
        _table_body,
        out_shape=jax.ShapeDtypeStruct((_TBL, _HID), jnp.float32),
    )(w1p, b1r, w2p, b2r)
    return out[:, 0]


def _worker_bounds(wid):
    base = _NCHUNK // _NW
    extra = _NCHUNK - base * _NW
    start = wid * base + jnp.minimum(wid, extra)
    cnt = base + jnp.where(wid < extra, 1, 0)
    return start, cnt


def _fill_zeros(zbuf):
    zeros16 = jnp.zeros((16,), jnp.float32)
    for i in range(_ZC // 16):
        zbuf[pl.ds(i * 16, 16)] = zeros16


def _zero_acc(acc, zbuf, sid):
    for j in range((_NZC + 15) // 16):
        k = sid + j * 16
        @pl.when(k < _NZC)
        def _():
            pltpu.sync_copy(zbuf, acc.at[pl.ds(k * _ZC, _ZC)])


def _write_partial(acc, zbuf, out_hbm, cid, sid):
    for j in range((_NZC + 15) // 16):
        k = sid + j * 16
        @pl.when(k < _NZC)
        def _():
            pltpu.sync_copy(acc.at[pl.ds(k * _ZC, _ZC)], zbuf)
            pltpu.sync_copy(zbuf, out_hbm.at[pl.ds(cid * _N_NODES + k * _ZC, _ZC)])


def _edge_pass(loads_hbm, loads_v, valsb, acc, wid, gather_vals, lsem, ssem):
    start, cnt = _worker_bounds(wid)

    def issue_loads(ci, p):
        r0 = (start + ci) * _CHUNK
        for h, v in zip(loads_hbm, loads_v[p]):
            pltpu.async_copy(h.at[pl.ds(r0, _CHUNK)], v, lsem)

    def wait_loads(p):
        for h, v in zip(loads_hbm, loads_v[p]):
            pltpu.make_async_copy(h.at[pl.ds(0, _CHUNK)], v, lsem).wait()

    def issue_scatters(p):
        dstb = loads_v[p][1]
        for j in range(_CHUNK):
            pltpu.async_copy(valsb[p].at[j], acc.at[dstb.at[j]], ssem,
                             add=True)

    def wait_scatters(p):
        dstb = loads_v[p][1]
        for j in range(_CHUNK):
            pltpu.make_async_copy(valsb[p].at[j], acc.at[dstb.at[j]],
                                  ssem).wait()

    issue_loads(0, 0)

    def pair_body(q, _):
        for p in (0, 1):
            ci = q * 2 + p
            @pl.when(ci < cnt)
            def _():
                wait_loads(p)
                @pl.when(ci > 0)
                def _():
                    wait_scatters(1 - p)
                @pl.when(ci + 1 < cnt)
                def _():
                    issue_loads(ci + 1, 1 - p)
                gather_vals(p)
                issue_scatters(p)
        return 0

    lax.fori_loop(0, (cnt + 1) // 2, pair_body, 0)

    @pl.when(lax.rem(cnt, 2) == 1)
    def _():
        wait_scatters(0)
    @pl.when(lax.rem(cnt, 2) == 0)
    def _():
        wait_scatters(1)


def _mega_kernel(nt_hbm, src_hbm, dst_hbm, et_hbm, tbl_hbm, nonce_hbm,
                 x3_hbm, x1_hbm, q_hbm, p2_hbm, p3_hbm, flags_hbm,
                 big_v, tbl_v,
                 srcb0, dstb0, etb0, srcb1, dstb1, etb1,
                 valsb0, valsb1,
                 t0a, t1a, t2a, t0b, t1b, t2b,
                 zbuf, acc, nbuf, fbuf, fbuf2, lsem, ssem):
    cid = lax.axis_index("c")
    sid = lax.axis_index("s")
    wid = sid * 2 + cid

    pltpu.sync_copy(nonce_hbm, nbuf)

    def xbar(b):
        plsc.subcore_barrier()
        target = nbuf[pl.ds(0, 16)][b]
        @pl.when(sid == 0)
        def _():
            fbuf[pl.ds(0, 16)] = jnp.full((16,), 0, jnp.int32) + target
            pltpu.sync_copy(fbuf, flags_hbm.at[pl.ds(cid * 16, 16)])

        def cond(v):
            return v != target

        def body(v):
            pltpu.sync_copy(flags_hbm.at[pl.ds((1 - cid) * 16, 16)], fbuf2)
            return fbuf2[pl.ds(0, 16)][0]

        lax.while_loop(cond, body, target - 1)

    loads3 = [[srcb0, dstb0, etb0], [srcb1, dstb1, etb1]]
    loads2 = [[srcb0, dstb0], [srcb1, dstb1]]
    valsb = [valsb0, valsb1]
    stage_bufs = [[t0a, t1a, t2a], [t0b, t1b, t2b]]

    pltpu.sync_copy(nt_hbm, big_v)
    pltpu.sync_copy(tbl_hbm, tbl_v)
    _fill_zeros(zbuf)
    _zero_acc(acc, zbuf, sid)
    plsc.subcore_barrier()

    def gather_vals1(p):
        srcb, dstb, etb = loads3[p]
        for j in range(_CHUNK):
            for b in range(_LANES // 16):
                s16 = srcb[j, pl.ds(b * 16, 16)]
                d16 = dstb[j, pl.ds(b * 16, 16)]
                e16 = etb[j, pl.ds(b * 16, 16)]
                ht = plsc.load_gather(big_v, [s16])
                tt = plsc.load_gather(big_v, [d16])
                combo = e16 * 16 + ht * 4 + tt
                valsb[p][j, pl.ds(b * 16, 16)] = plsc.load_gather(tbl_v, [combo])

    _edge_pass([src_hbm, dst_hbm, et_hbm], loads3, valsb, acc, wid,
               gather_vals1, lsem, ssem)
    plsc.subcore_barrier()
    _write_partial(acc, zbuf, q_hbm, cid, sid)
    xbar(0)

    def stage(parts, emit_ref):
        def stage_issue(k, p):
            bufs = stage_bufs[p]
            for h, v in zip(parts, bufs):
                pltpu.async_copy(h.at[pl.ds(k * _ZC, _ZC)], v, lsem)

        def stage_wait(p):
            bufs = stage_bufs[p]
            for h, v in zip(parts, bufs):
                pltpu.make_async_copy(h.at[pl.ds(0, _ZC)], v, lsem).wait()

        stage_issue(0, 0)

        def stage_body(qq, _):
            for p in (0, 1):
                k = qq * 2 + p
                @pl.when(k < _NZC)
                def _():
                    stage_wait(p)
                    @pl.when(k + 1 < _NZC)
                    def _():
                        stage_issue(k + 1, 1 - p)
                    bufs = stage_bufs[p]
                    for i in range(_ZC // 16):
                        v = bufs[0][pl.ds(i * 16, 16)]
                        for h in range(1, len(parts)):
                            v = v + bufs[h][pl.ds(i * 16, 16)]
                        big_v[pl.ds(k * _ZC + i * 16, 16)] = plsc.bitcast(
                            v, jnp.int32)
                    if emit_ref is not None:
                        @pl.when(lax.rem(k, _NW) == wid)
                        def _():
                            for i in range(_ZC // 16):
                                v = bufs[0][pl.ds(i * 16, 16)]
                                for h in range(1, len(parts)):
                                    v = v + bufs[h][pl.ds(i * 16, 16)]
                                zbuf[pl.ds(i * 16, 16)] = v
                            pltpu.sync_copy(
                                zbuf, emit_ref.at[pl.ds(k * _ZC, _ZC)])
            return 0

        lax.fori_loop(0, (_NZC + 1) // 2, stage_body, 0)

    def gather_vals2(p):
        srcb = loads2[p][0]
        for j in range(_CHUNK):
            for b in range(_LANES // 16):
                s16 = srcb[j, pl.ds(b * 16, 16)]
                g = plsc.load_gather(big_v, [s16])
                valsb[p][j, pl.ds(b * 16, 16)] = plsc.bitcast(g, jnp.float32)

    stage([q_hbm.at[pl.ds(0, _N_NODES)], q_hbm.at[pl.ds(_N_NODES, _N_NODES)]],
          x1_hbm)
    _fill_zeros(zbuf)
    _zero_acc(acc, zbuf, sid)
    plsc.subcore_barrier()
    _edge_pass([src_hbm, dst_hbm], loads2, valsb, acc, wid, gather_vals2,
               lsem, ssem)
    plsc.subcore_barrier()
    _write_partial(acc, zbuf, p2_hbm, cid, sid)
    xbar(1)

    stage([p2_hbm.at[pl.ds(0, _N_NODES)],
           p2_hbm.at[pl.ds(_N_NODES, _N_NODES)], x1_hbm], None)
    _fill_zeros(zbuf)
    _zero_acc(acc, zbuf, sid)
    plsc.subcore_barrier()
    _edge_pass([src_hbm, dst_hbm], loads2, valsb, acc, wid, gather_vals2,
               lsem, ssem)
    plsc.subcore_barrier()
    _write_partial(acc, zbuf, p3_hbm, cid, sid)
    xbar(2)

    for j in range((_NZC + _NW - 1) // _NW):
        k = wid + j * _NW
        @pl.when(k < _NZC)
        def _():
            pltpu.sync_copy(p3_hbm.at[pl.ds(k * _ZC, _ZC)], t0a)
            pltpu.sync_copy(p3_hbm.at[pl.ds(_N_NODES + k * _ZC, _ZC)], t1a)
            pltpu.sync_copy(x1_hbm.at[pl.ds(k * _ZC, _ZC)], t2a)
            for i in range(_ZC // 16):
                t0b[pl.ds(i * 16, 16)] = (t0a[pl.ds(i * 16, 16)]
                                          + t1a[pl.ds(i * 16, 16)]
                                          + t2a[pl.ds(i * 16, 16)])
            pltpu.sync_copy(t0b, x3_hbm.at[pl.ds(k * _ZC, _ZC)])


@jax.jit
def kernel(node_type, edge_index, edge_type, W1, b1, W2, b2):
    table = _make_table(W1, b1, W2, b2)

    src2d = edge_index[0].reshape(_ROWS, _LANES)
    dst2d = edge_index[1].reshape(_ROWS, _LANES)
    et2d = edge_type.reshape(_ROWS, _LANES)

    f32 = jnp.float32
    i32 = jnp.int32

    def ebuf(dt=i32):
        return pltpu.VMEM((_CHUNK, _LANES), dt)

    mega = pl.kernel(
        _mega_kernel,
        out_type=(jax.ShapeDtypeStruct((_N_NODES,), f32),
                  jax.ShapeDtypeStruct((_N_NODES,), f32),
                  jax.ShapeDtypeStruct((2 * _N_NODES,), f32),
                  jax.ShapeDtypeStruct((2 * _N_NODES,), f32),
                  jax.ShapeDtypeStruct((2 * _N_NODES,), f32),
                  jax.ShapeDtypeStruct((32,), i32)),
        mesh=plsc.VectorSubcoreMesh(core_axis_name="c", subcore_axis_name="s"),
        compiler_params=pltpu.CompilerParams(needs_layout_passes=False),
        scratch_types=(
            [pltpu.VMEM((_N_NODES,), i32),
             pltpu.VMEM((_TBL,), f32)]
            + [ebuf() for _ in range(6)]
            + [ebuf(f32) for _ in range(2)]
            + [pltpu.VMEM((_ZC,), f32) for _ in range(6)]
            + [pltpu.VMEM((_ZC,), f32),
               pltpu.VMEM_SHARED((_N_NODES,), f32),
               pltpu.VMEM((16,), i32),
               pltpu.VMEM((16,), i32),
               pltpu.VMEM((16,), i32),
               pltpu.SemaphoreType.DMA,
               pltpu.SemaphoreType.DMA]
        ),
    )
    mix = (edge_index[0, 0] * jnp.int32(-1640531527)
           + edge_index[1, 1] * jnp.int32(40503)
           + edge_type[2] * jnp.int32(2654435769 - 2**32)
           + node_type[0] + jnp.int32(12345))
    nonce_arr = mix + jnp.arange(16, dtype=i32)
    x3, _, _, _, _, _ = mega(node_type, src2d, dst2d, et2d, table, nonce_arr)
    return x3.reshape(_N_NODES, 1)
